# baseline (speedup 1.0000x reference)
import functools

import jax
import jax.numpy as jnp
from jax import lax
from jax.experimental import pallas as pl
from jax.experimental.pallas import tpu as pltpu
from jax.experimental.pallas import tpu_sc as plsc

_K = 10
_NKP = 9
_C = 1024
_BIG = 1e30


def _sc_scan_body(n_points, kpts_hbm, cpt_hbm, pcld_hbm, seg_hbm, out_hbm,
                  kc, cc, sgc, gbuf, st, sem):
    b = lax.axis_index("s") * 2 + lax.axis_index("c")
    base_pt = b * n_points
    it = lax.iota(jnp.int32, 16)
    it24 = it * 24
    it3 = it * 3
    it2 = it * 2
    big = jnp.float32(_BIG)

    def chunk_body(q, carry):
        s_pt = base_pt + q * _C
        pltpu.sync_copy(kpts_hbm.at[pl.ds(s_pt * 24, _C * 24)], kc)
        pltpu.sync_copy(cpt_hbm.at[pl.ds(s_pt * 3, _C * 3)], cc)
        pltpu.sync_copy(seg_hbm.at[pl.ds(s_pt * 2, _C * 2)], sgc)

        def group_body(j, carry):
            bvs, bis, thrs = carry
            p2 = j * 32 + it2
            gp = q * _C + j * 16 + it
            s0 = plsc.load_gather(sgc, [p2])
            s1 = plsc.load_gather(sgc, [p2 + 1])
            obj = s1 > s0
            sent = jnp.float32(1e18) + gp.astype(jnp.float32) * jnp.float32(1e12)
            r24 = j * 384 + it24
            r3 = j * 48 + it3
            nbv, nbi, nth = [], [], []
            for kk in range(_NKP):
                if kk < 8:
                    base = r24 + kk * 3
                    x = plsc.load_gather(kc, [base])
                    y = plsc.load_gather(kc, [base + 1])
                    z = plsc.load_gather(kc, [base + 2])
                else:
                    x = plsc.load_gather(cc, [r3])
                    y = plsc.load_gather(cc, [r3 + 1])
                    z = plsc.load_gather(cc, [r3 + 2])
                n2 = x * x + y * y + z * z
                key = jnp.where(obj, n2, sent)
                m = jnp.min(key)
                bv, bi, th = bvs[kk], bis[kk], thrs[kk]

                def merge(bv=bv, bi=bi, key=key):
                    sk, sv = plsc.sort_key_val(key, gp)
                    rk = lax.rev(sk, (0,))
                    rv = lax.rev(sv, (0,))
                    take = bv <= rk
                    lo_k = jnp.where(take, bv, rk)
                    lo_v = jnp.where(take, bi, rv)
                    nk, nv = plsc.sort_key_val(lo_k, lo_v)
                    t = jnp.min(jnp.where(it == 9, nk, big))
                    return nk, nv, t

                def keep(bv=bv, bi=bi, th=th):
                    return bv, bi, th

                res = lax.cond(m < th, merge, keep)
                nbv.append(res[0])
                nbi.append(res[1])
                nth.append(res[2])
            return tuple(nbv), tuple(nbi), tuple(nth)

        return lax.fori_loop(0, _C // 16, group_body, carry)

    init = (tuple(jnp.full((16,), big, jnp.float32) for _ in range(_NKP)),
            tuple(jnp.zeros((16,), jnp.int32) for _ in range(_NKP)),
            tuple(jnp.float32(_BIG) for _ in range(_NKP)))
    _, bis, _ = lax.fori_loop(0, n_points // _C, chunk_body, init)

    copies = []
    for kk in range(_NKP):
        bi = bis[kk] + base_pt
        for c in range(3):
            if kk < 8:
                src = kpts_hbm.at[bi * 24 + kk * 3 + c]
            else:
                src = cpt_hbm.at[bi * 3 + c]
            j = kk * 6 + c * 2
            copies.append(pltpu.async_copy(src, gbuf.at[pl.ds(j * 16, 16)], sem))
            copies.append(pltpu.async_copy(pcld_hbm.at[bi * 3 + c],
                                           gbuf.at[pl.ds((j + 1) * 16, 16)], sem))
    for cp in copies:
        cp.wait()
    for kk in range(_NKP):
        for c in range(3):
            j = kk * 6 + c * 2
            cand = gbuf[pl.ds(j * 16, 16)] + gbuf[pl.ds((j + 1) * 16, 16)]
            st[pl.ds((c * _NKP + kk) * 16, 16)] = cand
    pltpu.sync_copy(st, out_hbm.at[b])


def _cluster_kernel(g_ref, o_ref):
    g = g_ref[...]
    third = g.shape[0] // 3
    x = g[:third]
    y = g[third:2 * third]
    z = g[2 * third:]
    valid = lax.broadcasted_iota(jnp.int32, x.shape, 1) < _K
    inv_k = jnp.float32(1.0 / _K)
    eps = jnp.float32(1e-9)

    def stats(v):
        mean = jnp.sum(jnp.where(valid, v, 0.0), axis=1, keepdims=True) * inv_k
        d = v - mean
        std = jnp.sqrt(
            jnp.sum(jnp.where(valid, d * d, 0.0), axis=1, keepdims=True) * inv_k)
        return d, std

    dx, sx = stats(x)
    dy, sy = stats(y)
    dz, sz = stats(z)
    inl = (valid & (jnp.abs(dx) <= sx + eps) & (jnp.abs(dy) <= sy + eps)
           & (jnp.abs(dz) <= sz + eps))
    w = inl.astype(jnp.float32)
    denom = jnp.sum(w, axis=1, keepdims=True) + jnp.float32(1e-8)
    ox = jnp.sum(jnp.where(valid, x, 0.0) * w, axis=1, keepdims=True) / denom
    oy = jnp.sum(jnp.where(valid, y, 0.0) * w, axis=1, keepdims=True) / denom
    oz = jnp.sum(jnp.where(valid, z, 0.0) * w, axis=1, keepdims=True) / denom
    o_ref[...] = jnp.concatenate([ox, oy, oz], axis=1)


def kernel(pcld_input, kpts_pre_input, cpt_pre_input, seg_pre_input):
    b, n = pcld_input.shape[0], pcld_input.shape[1]
    kpts_r = kpts_pre_input.reshape(b * n * 24)
    cpt_r = cpt_pre_input.reshape(b * n * 3)
    pcld_r = pcld_input.reshape(b * n * 3)
    seg_r = seg_pre_input.reshape(b * n * 2)

    mesh = plsc.VectorSubcoreMesh(core_axis_name="c", subcore_axis_name="s")
    scan = pl.kernel(
        functools.partial(_sc_scan_body, n),
        mesh=mesh,
        compiler_params=pltpu.CompilerParams(needs_layout_passes=False),
        out_type=jax.ShapeDtypeStruct((b, 3 * _NKP * 16), jnp.float32),
        scratch_types=[
            pltpu.VMEM((_C * 24,), jnp.float32),
            pltpu.VMEM((_C * 3,), jnp.float32),
            pltpu.VMEM((_C * 2,), jnp.float32),
            pltpu.VMEM((_NKP * 6 * 16,), jnp.float32),
            pltpu.VMEM((3 * _NKP * 16,), jnp.float32),
            pltpu.SemaphoreType.DMA,
        ],
    )
    g = scan(kpts_r, cpt_r, pcld_r, seg_r)
    g2 = g.reshape(b, 3, _NKP, 16).transpose(1, 0, 2, 3).reshape(3 * b * _NKP, 16)

    out = pl.pallas_call(
        _cluster_kernel,
        out_shape=jax.ShapeDtypeStruct((b * _NKP, 3), jnp.float32),
    )(g2)
    return out.reshape(b, _NKP, 3)


# speedup vs baseline: 16.6365x; 16.6365x over previous
import functools

import jax
import jax.numpy as jnp
from jax import lax
from jax.experimental import pallas as pl
from jax.experimental.pallas import tpu as pltpu
from jax.experimental.pallas import tpu_sc as plsc

_K = 10
_NKP = 9
_C = 1024
_BIG = 1e30


def _sc_scan_body(n_points, kpts_hbm, cpt_hbm, pcld_hbm, seg_hbm, out_hbm,
                  kc, cc, pc, sgc, st):
    b = lax.axis_index("s") * 2 + lax.axis_index("c")
    it = lax.iota(jnp.int32, 16)
    big = jnp.float32(_BIG)

    def chunk_body(q, carry):
        s = q * _C
        pltpu.sync_copy(kpts_hbm.at[b, :, :, pl.ds(s, _C)], kc)
        pltpu.sync_copy(cpt_hbm.at[b, :, pl.ds(s, _C)], cc)
        pltpu.sync_copy(pcld_hbm.at[:, b, pl.ds(s, _C)], pc)
        pltpu.sync_copy(seg_hbm.at[b, :, pl.ds(s, _C)], sgc)

        def group_body(j, carry):
            bks, bxs, bys, bzs, thrs = carry
            sl = pl.ds(j * 16, 16)
            gp = q * _C + j * 16 + it
            s0 = sgc[0, sl]
            s1 = sgc[1, sl]
            obj = s1 > s0
            sent = jnp.float32(1e18) + gp.astype(jnp.float32) * jnp.float32(1e12)
            p0 = pc[0, sl]
            p1 = pc[1, sl]
            p2 = pc[2, sl]
            nbk, nbx, nby, nbz, nth = [], [], [], [], []
            for kk in range(_NKP):
                if kk < 8:
                    x = kc[0, kk, sl]
                    y = kc[1, kk, sl]
                    z = kc[2, kk, sl]
                else:
                    x = cc[0, sl]
                    y = cc[1, sl]
                    z = cc[2, sl]
                n2 = x * x + y * y + z * z
                key = jnp.where(obj, n2, sent)
                m = jnp.min(key)
                bk, bx, by, bz, th = bks[kk], bxs[kk], bys[kk], bzs[kk], thrs[kk]

                def merge(bk=bk, bx=bx, by=by, bz=bz, key=key, x=x, y=y, z=z):
                    cx = p0 + x
                    cy = p1 + y
                    cz = p2 + z
                    sk, sx = plsc.sort_key_val(key, cx)
                    _, sy = plsc.sort_key_val(key, cy)
                    _, sz = plsc.sort_key_val(key, cz)
                    rk = lax.rev(sk, (0,))
                    take = bk <= rk
                    lo_k = jnp.where(take, bk, rk)
                    lo_x = jnp.where(take, bx, lax.rev(sx, (0,)))
                    lo_y = jnp.where(take, by, lax.rev(sy, (0,)))
                    lo_z = jnp.where(take, bz, lax.rev(sz, (0,)))
                    nk, nx = plsc.sort_key_val(lo_k, lo_x)
                    _, ny = plsc.sort_key_val(lo_k, lo_y)
                    _, nz = plsc.sort_key_val(lo_k, lo_z)
                    t = jnp.min(jnp.where(it == 9, nk, big))
                    return nk, nx, ny, nz, t

                def keep(bk=bk, bx=bx, by=by, bz=bz, th=th):
                    return bk, bx, by, bz, th

                res = lax.cond(m < th, merge, keep)
                nbk.append(res[0])
                nbx.append(res[1])
                nby.append(res[2])
                nbz.append(res[3])
                nth.append(res[4])
            return (tuple(nbk), tuple(nbx), tuple(nby), tuple(nbz), tuple(nth))

        return lax.fori_loop(0, _C // 16, group_body, carry)

    init = (tuple(jnp.full((16,), big, jnp.float32) for _ in range(_NKP)),
            tuple(jnp.zeros((16,), jnp.float32) for _ in range(_NKP)),
            tuple(jnp.zeros((16,), jnp.float32) for _ in range(_NKP)),
            tuple(jnp.zeros((16,), jnp.float32) for _ in range(_NKP)),
            tuple(jnp.float32(_BIG) for _ in range(_NKP)))
    _, bxs, bys, bzs, _ = lax.fori_loop(0, n_points // _C, chunk_body, init)

    for kk in range(_NKP):
        st[pl.ds(kk * 16, 16)] = bxs[kk]
        st[pl.ds((_NKP + kk) * 16, 16)] = bys[kk]
        st[pl.ds((2 * _NKP + kk) * 16, 16)] = bzs[kk]
    pltpu.sync_copy(st, out_hbm.at[b])


def _cluster_kernel(g_ref, o_ref):
    g = g_ref[...]
    third = g.shape[0] // 3
    x = g[:third]
    y = g[third:2 * third]
    z = g[2 * third:]
    valid = lax.broadcasted_iota(jnp.int32, x.shape, 1) < _K
    inv_k = jnp.float32(1.0 / _K)
    eps = jnp.float32(1e-9)

    def stats(v):
        mean = jnp.sum(jnp.where(valid, v, 0.0), axis=1, keepdims=True) * inv_k
        d = v - mean
        std = jnp.sqrt(
            jnp.sum(jnp.where(valid, d * d, 0.0), axis=1, keepdims=True) * inv_k)
        return d, std

    dx, sx = stats(x)
    dy, sy = stats(y)
    dz, sz = stats(z)
    inl = (valid & (jnp.abs(dx) <= sx + eps) & (jnp.abs(dy) <= sy + eps)
           & (jnp.abs(dz) <= sz + eps))
    w = inl.astype(jnp.float32)
    denom = jnp.sum(w, axis=1, keepdims=True) + jnp.float32(1e-8)
    ox = jnp.sum(jnp.where(valid, x, 0.0) * w, axis=1, keepdims=True) / denom
    oy = jnp.sum(jnp.where(valid, y, 0.0) * w, axis=1, keepdims=True) / denom
    oz = jnp.sum(jnp.where(valid, z, 0.0) * w, axis=1, keepdims=True) / denom
    o_ref[...] = jnp.concatenate([ox, oy, oz], axis=1)


def kernel(pcld_input, kpts_pre_input, cpt_pre_input, seg_pre_input):
    b, n = pcld_input.shape[0], pcld_input.shape[1]
    kpts_t = jnp.transpose(kpts_pre_input, (0, 3, 2, 1))
    cpt_t = jnp.transpose(cpt_pre_input, (0, 3, 2, 1)).reshape(b, 3, n)
    pcld_t = jnp.transpose(pcld_input, (2, 0, 1))
    seg_t = jnp.transpose(seg_pre_input, (0, 2, 1))

    mesh = plsc.VectorSubcoreMesh(core_axis_name="c", subcore_axis_name="s")
    scan = pl.kernel(
        functools.partial(_sc_scan_body, n),
        mesh=mesh,
        compiler_params=pltpu.CompilerParams(needs_layout_passes=False),
        out_type=jax.ShapeDtypeStruct((b, 3 * _NKP * 16), jnp.float32),
        scratch_types=[
            pltpu.VMEM((3, 8, _C), jnp.float32),
            pltpu.VMEM((3, _C), jnp.float32),
            pltpu.VMEM((3, _C), jnp.float32),
            pltpu.VMEM((2, _C), jnp.float32),
            pltpu.VMEM((3 * _NKP * 16,), jnp.float32),
        ],
    )
    g = scan(kpts_t, cpt_t, pcld_t, seg_t)
    g2 = g.reshape(b, 3, _NKP, 16).transpose(1, 0, 2, 3).reshape(3 * b * _NKP, 16)

    out = pl.pallas_call(
        _cluster_kernel,
        out_shape=jax.ShapeDtypeStruct((b * _NKP, 3), jnp.float32),
    )(g2)
    return out.reshape(b, _NKP, 3)
